# Initial kernel scaffold; baseline (speedup 1.0000x reference)
import jax
import jax.numpy as jnp
from jax.experimental import pallas as pl
from jax.experimental.pallas import tpu as pltpu

_NUM_NODES = 1024
_INPUT_SIZE = 512
_OUTPUT_SIZE = 128
_FAN_IN = 16
_BATCH = 2048
_SUB = 8
_LANE = 256


def _neagent_kernel(idx_ref, w_ref, x_ref, out_ref, a_ref):
    a_ref[0:_INPUT_SIZE] = x_ref[...]

    def body(i, carry):
        acc = a_ref[pl.ds(idx_ref[i, 0], 1)] * w_ref[i, 0]
        for j in range(1, _FAN_IN):
            acc = acc + a_ref[pl.ds(idx_ref[i, j], 1)] * w_ref[i, j]
        a_ref[pl.ds(_INPUT_SIZE + i, 1)] = jnp.tanh(acc)
        return carry

    jax.lax.fori_loop(0, _NUM_NODES, body, 0)
    out_ref[...] = a_ref[_INPUT_SIZE + _NUM_NODES - _OUTPUT_SIZE:]


def kernel(x, in_idxs, weights):
    xT = x.T.reshape(_INPUT_SIZE, _SUB, _LANE)
    idx = in_idxs.astype(jnp.int32)
    out = pl.pallas_call(
        _neagent_kernel,
        out_shape=jax.ShapeDtypeStruct((_OUTPUT_SIZE, _SUB, _LANE), jnp.float32),
        in_specs=[
            pl.BlockSpec(memory_space=pltpu.SMEM),
            pl.BlockSpec(memory_space=pltpu.SMEM),
            pl.BlockSpec(memory_space=pltpu.VMEM),
        ],
        out_specs=pl.BlockSpec(memory_space=pltpu.VMEM),
        scratch_shapes=[
            pltpu.VMEM((_INPUT_SIZE + _NUM_NODES, _SUB, _LANE), jnp.float32)
        ],
    )(idx, weights, xT)
    return out.reshape(_OUTPUT_SIZE, _BATCH)


# speedup vs baseline: 183.9180x; 183.9180x over previous
import jax
import jax.numpy as jnp
from jax.experimental import pallas as pl
from jax.experimental.pallas import tpu as pltpu

_NUM_NODES = 1024
_INPUT_SIZE = 512
_OUTPUT_SIZE = 128
_FAN_IN = 16
_BATCH = 2048
_SUB = 8
_LANE = 256


def _neagent_kernel(idx_ref, w_ref, x_ref, out_ref, a_ref):
    a_ref[0:_INPUT_SIZE] = x_ref[...]

    def body(i, carry):
        base = i * _FAN_IN
        acc = None
        for j in range(_FAN_IN):
            t = a_ref[pl.ds(idx_ref[base + j], 1)] * w_ref[base + j]
            acc = t if acc is None else acc + t
        a_ref[pl.ds(_INPUT_SIZE + i, 1)] = jnp.tanh(acc)
        return carry

    jax.lax.fori_loop(0, _NUM_NODES, body, 0)
    out_ref[...] = a_ref[_INPUT_SIZE + _NUM_NODES - _OUTPUT_SIZE:]


def kernel(x, in_idxs, weights):
    xT = x.T.reshape(_INPUT_SIZE, _SUB, _LANE)
    idx = in_idxs.astype(jnp.int32).reshape(-1)
    w_flat = weights.reshape(-1)
    out = pl.pallas_call(
        _neagent_kernel,
        out_shape=jax.ShapeDtypeStruct((_OUTPUT_SIZE, _SUB, _LANE), jnp.float32),
        in_specs=[
            pl.BlockSpec(memory_space=pltpu.SMEM),
            pl.BlockSpec(memory_space=pltpu.SMEM),
            pl.BlockSpec(memory_space=pltpu.VMEM),
        ],
        out_specs=pl.BlockSpec(memory_space=pltpu.VMEM),
        scratch_shapes=[
            pltpu.VMEM((_INPUT_SIZE + _NUM_NODES, _SUB, _LANE), jnp.float32)
        ],
    )(idx, w_flat, xT)
    return out.reshape(_OUTPUT_SIZE, _BATCH)
